# baseline (speedup 1.0000x reference)
import functools
import numpy as np
import jax
import jax.numpy as jnp
from jax import lax
from jax.experimental import pallas as pl
from jax.experimental.pallas import tpu as pltpu
from jax.experimental.pallas import tpu_sc as plsc

D_MODEL = 256
N_ROWS = 16384
NW = 32
RPW = N_ROWS // NW
CHUNK = 128

_c = np.arange(128)
_FREQ = 1.0 / (10000.0 ** (2.0 * (_c // 2) / 128.0))
_PH = (_c % 2) * (np.pi / 2.0)
_PE = np.sin(np.arange(64)[:, None] * _FREQ[None, :] + _PH[None, :]).astype(np.float32)

_mesh = plsc.VectorSubcoreMesh(core_axis_name="c", subcore_axis_name="s")


@functools.partial(
    pl.kernel,
    mesh=_mesh,
    out_type=[jax.ShapeDtypeStruct((N_ROWS, D_MODEL), jnp.float32),
              jax.ShapeDtypeStruct((NW, D_MODEL), jnp.float32)],
    scratch_types=[
        pltpu.VMEM((256,), jnp.float32),
        pltpu.VMEM((RPW,), jnp.int32),
        pltpu.VMEM((RPW,), jnp.int32),
        pltpu.VMEM((RPW,), jnp.int32),
        pltpu.VMEM((RPW,), jnp.int32),
        pltpu.VMEM((RPW + 16,), jnp.float32),
        pltpu.VMEM((RPW + 16,), jnp.float32),
        pltpu.VMEM((CHUNK, 128), jnp.float32),
        pltpu.VMEM((CHUNK, 128), jnp.float32),
        pltpu.VMEM((CHUNK, 128), jnp.float32),
        pltpu.VMEM((CHUNK, 128), jnp.float32),
        pltpu.VMEM((CHUNK, D_MODEL), jnp.float32),
        pltpu.VMEM((1, D_MODEL), jnp.float32),
        pltpu.SemaphoreType.DMA,
    ],
)
def _sc_embed(g_hbm, r_hbm, t_hbm, q_hbm, pv_hbm, hp_hbm,
              gate_hbm, pe_hbm, roleb_hbm, pirow_hbm,
              out_hbm, psum_hbm,
              pirow_v, gb, rb, tb, qb, pvb, hpb,
              grow, trow, qrow, rrow, outbuf, psum_v, sem):
    wid = lax.axis_index("s") * 2 + lax.axis_index("c")
    base = wid * RPW

    pltpu.sync_copy(pirow_hbm, pirow_v)
    pltpu.sync_copy(g_hbm.at[pl.ds(base, RPW)], gb)
    pltpu.sync_copy(r_hbm.at[pl.ds(base, RPW)], rb)
    pltpu.sync_copy(t_hbm.at[pl.ds(base, RPW)], tb)
    pltpu.sync_copy(q_hbm.at[pl.ds(base, RPW)], qb)
    pltpu.sync_copy(pv_hbm.at[pl.ds(base, RPW)], pvb.at[pl.ds(0, RPW)])
    pltpu.sync_copy(hp_hbm.at[pl.ds(base, RPW)], hpb.at[pl.ds(0, RPW)])

    prow_k = [pirow_v[pl.ds(16 * k, 16)] for k in range(8)]
    irow_k = [pirow_v[pl.ds(128 + 16 * k, 16)] for k in range(8)]
    zero16 = jnp.zeros((16,), jnp.float32)

    acc = tuple([zero16] * 16)
    for c in range(RPW // CHUNK):
        o = c * CHUNK
        d1 = pltpu.async_copy(gate_hbm.at[gb.at[pl.ds(o, CHUNK)]], grow, sem)
        d2 = pltpu.async_copy(pe_hbm.at[tb.at[pl.ds(o, CHUNK)]], trow, sem)
        d3 = pltpu.async_copy(pe_hbm.at[qb.at[pl.ds(o, CHUNK)]], qrow, sem)
        d4 = pltpu.async_copy(roleb_hbm.at[rb.at[pl.ds(o, CHUNK)]], rrow, sem)
        d1.wait(); d2.wait(); d3.wait(); d4.wait()

        @plsc.parallel_loop(0, CHUNK, step=1, unroll=4, carry=acc)
        def rowloop(nloc, acc):
            n = o + nloc
            pvf = jnp.full((16,), pvb[pl.ds(n, 16)][0], jnp.float32)
            hpf = jnp.full((16,), hpb[pl.ds(n, 16)][0], jnp.float32)
            newacc = []
            for k in range(8):
                v = (grow[nloc, pl.ds(16 * k, 16)]
                     + trow[nloc, pl.ds(16 * k, 16)])
                outbuf[nloc, pl.ds(16 * k, 16)] = v
                newacc.append(acc[k] + v)
            for k in range(8):
                v = (rrow[nloc, pl.ds(16 * k, 16)]
                     + qrow[nloc, pl.ds(16 * k, 16)]
                     + pvf * prow_k[k]
                     + hpf * irow_k[k])
                outbuf[nloc, pl.ds(128 + 16 * k, 16)] = v
                newacc.append(acc[8 + k] + v)
            return tuple(newacc)

        acc = rowloop
        pltpu.sync_copy(outbuf, out_hbm.at[pl.ds(base + o, CHUNK)])

    for k in range(16):
        psum_v[0, pl.ds(16 * k, 16)] = acc[k]
    pltpu.sync_copy(psum_v, psum_hbm.at[pl.ds(wid, 1)])


def kernel(gate_idx, role_idx, param_val, has_param, grid_positions,
           gate_table, role_table, W_param, b_param):
    roleb = jnp.zeros((4, 128), jnp.float32)
    roleb = roleb.at[:, 0:64].set(role_table)
    roleb = roleb.at[:, 64:127].set(jnp.broadcast_to(b_param[None, :], (4, 63)))
    pirow = jnp.zeros((256,), jnp.float32)
    pirow = pirow.at[64:127].set(W_param[0])
    pirow = pirow.at[255].set(1.0)

    node_embeddings, psum = _sc_embed(
        gate_idx.astype(jnp.int32),
        role_idx.astype(jnp.int32),
        grid_positions[:, 0].astype(jnp.int32),
        grid_positions[:, 1].astype(jnp.int32),
        param_val, has_param,
        gate_table, jnp.asarray(_PE), roleb, pirow)

    circuit_embedding = psum.sum(axis=0) * (1.0 / N_ROWS)
    return node_embeddings, circuit_embedding


# speedup vs baseline: 1.0104x; 1.0104x over previous
import functools
import numpy as np
import jax
import jax.numpy as jnp
from jax import lax
from jax.experimental import pallas as pl
from jax.experimental.pallas import tpu as pltpu
from jax.experimental.pallas import tpu_sc as plsc

D_MODEL = 256
N_ROWS = 16384
NW = 32
RPW = N_ROWS // NW
CHUNK = 128

_c = np.arange(128)
_FREQ = 1.0 / (10000.0 ** (2.0 * (_c // 2) / 128.0))
_PH = (_c % 2) * (np.pi / 2.0)
_PE = np.sin(np.arange(64)[:, None] * _FREQ[None, :] + _PH[None, :]).astype(np.float32)

_mesh = plsc.VectorSubcoreMesh(core_axis_name="c", subcore_axis_name="s")


@functools.partial(
    pl.kernel,
    mesh=_mesh,
    out_type=[jax.ShapeDtypeStruct((N_ROWS, D_MODEL), jnp.float32),
              jax.ShapeDtypeStruct((NW, D_MODEL), jnp.float32)],
    scratch_types=[
        pltpu.VMEM((256,), jnp.float32),
        pltpu.VMEM((RPW,), jnp.int32),
        pltpu.VMEM((RPW,), jnp.int32),
        pltpu.VMEM((RPW,), jnp.int32),
        pltpu.VMEM((RPW,), jnp.int32),
        pltpu.VMEM((RPW + 16,), jnp.float32),
        pltpu.VMEM((RPW + 16,), jnp.float32),
        pltpu.VMEM((CHUNK, 128), jnp.float32),
        pltpu.VMEM((CHUNK, 128), jnp.float32),
        pltpu.VMEM((CHUNK, 128), jnp.float32),
        pltpu.VMEM((CHUNK, 128), jnp.float32),
        pltpu.VMEM((CHUNK, D_MODEL), jnp.float32),
        pltpu.VMEM((1, D_MODEL), jnp.float32),
        pltpu.SemaphoreType.DMA,
    ],
)
def _sc_embed(g_hbm, r_hbm, t_hbm, q_hbm, pv_hbm, hp_hbm,
              gate_hbm, pe_hbm, roleb_hbm, pirow_hbm,
              out_hbm, psum_hbm,
              pirow_v, gb, rb, tb, qb, pvb, hpb,
              grow, trow, qrow, rrow, outbuf, psum_v, sem):
    wid = lax.axis_index("s") * 2 + lax.axis_index("c")
    base = wid * RPW

    pltpu.sync_copy(pirow_hbm, pirow_v)
    pltpu.sync_copy(g_hbm.at[pl.ds(base, RPW)], gb)
    pltpu.sync_copy(r_hbm.at[pl.ds(base, RPW)], rb)
    pltpu.sync_copy(t_hbm.at[pl.ds(base, RPW)], tb)
    pltpu.sync_copy(q_hbm.at[pl.ds(base, RPW)], qb)
    pltpu.sync_copy(pv_hbm.at[pl.ds(base, RPW)], pvb.at[pl.ds(0, RPW)])
    pltpu.sync_copy(hp_hbm.at[pl.ds(base, RPW)], hpb.at[pl.ds(0, RPW)])

    prow_k = [pirow_v[pl.ds(16 * k, 16)] for k in range(8)]
    irow_k = [pirow_v[pl.ds(128 + 16 * k, 16)] for k in range(8)]
    zero16 = jnp.zeros((16,), jnp.float32)

    acc = tuple([zero16] * 16)
    for c in range(RPW // CHUNK):
        o = c * CHUNK
        d1 = pltpu.async_copy(gate_hbm.at[gb.at[pl.ds(o, CHUNK)]], grow, sem)
        d2 = pltpu.async_copy(pe_hbm.at[tb.at[pl.ds(o, CHUNK)]], trow, sem)
        d3 = pltpu.async_copy(pe_hbm.at[qb.at[pl.ds(o, CHUNK)]], qrow, sem)
        d4 = pltpu.async_copy(roleb_hbm.at[rb.at[pl.ds(o, CHUNK)]], rrow, sem)
        d1.wait(); d2.wait(); d3.wait(); d4.wait()

        @plsc.parallel_loop(0, CHUNK, step=1, unroll=4, carry=acc)
        def rowloop(nloc, acc):
            n = o + nloc
            pvf = jnp.full((16,), pvb[pl.ds(n, 16)][0], jnp.float32)
            hpf = jnp.full((16,), hpb[pl.ds(n, 16)][0], jnp.float32)
            newacc = []
            for k in range(8):
                v = (grow[nloc, pl.ds(16 * k, 16)]
                     + trow[nloc, pl.ds(16 * k, 16)])
                outbuf[nloc, pl.ds(16 * k, 16)] = v
                newacc.append(acc[k] + v)
            for k in range(8):
                v = (rrow[nloc, pl.ds(16 * k, 16)]
                     + qrow[nloc, pl.ds(16 * k, 16)])
                if k >= 4:
                    v = v + pvf * prow_k[k]
                if k == 7:
                    v = v + hpf * irow_k[k]
                outbuf[nloc, pl.ds(128 + 16 * k, 16)] = v
                newacc.append(acc[8 + k] + v)
            return tuple(newacc)

        acc = rowloop
        pltpu.sync_copy(outbuf, out_hbm.at[pl.ds(base + o, CHUNK)])

    for k in range(16):
        psum_v[0, pl.ds(16 * k, 16)] = acc[k]
    pltpu.sync_copy(psum_v, psum_hbm.at[pl.ds(wid, 1)])


def kernel(gate_idx, role_idx, param_val, has_param, grid_positions,
           gate_table, role_table, W_param, b_param):
    roleb = jnp.zeros((4, 128), jnp.float32)
    roleb = roleb.at[:, 0:64].set(role_table)
    roleb = roleb.at[:, 64:127].set(jnp.broadcast_to(b_param[None, :], (4, 63)))
    pirow = jnp.zeros((256,), jnp.float32)
    pirow = pirow.at[64:127].set(W_param[0])
    pirow = pirow.at[255].set(1.0)

    node_embeddings, psum = _sc_embed(
        gate_idx.astype(jnp.int32),
        role_idx.astype(jnp.int32),
        grid_positions[:, 0].astype(jnp.int32),
        grid_positions[:, 1].astype(jnp.int32),
        param_val, has_param,
        gate_table, jnp.asarray(_PE), roleb, pirow)

    circuit_embedding = psum.sum(axis=0) * (1.0 / N_ROWS)
    return node_embeddings, circuit_embedding


# speedup vs baseline: 1.0329x; 1.0223x over previous
import functools
import numpy as np
import jax
import jax.numpy as jnp
from jax import lax
from jax.experimental import pallas as pl
from jax.experimental.pallas import tpu as pltpu
from jax.experimental.pallas import tpu_sc as plsc

D_MODEL = 256
N_ROWS = 16384
NW = 32
RPW = N_ROWS // NW
CHUNK = 128

_c = np.arange(128)
_FREQ = 1.0 / (10000.0 ** (2.0 * (_c // 2) / 128.0))
_PH = (_c % 2) * (np.pi / 2.0)
_PE = np.sin(np.arange(64)[:, None] * _FREQ[None, :] + _PH[None, :]).astype(np.float32)

_mesh = plsc.VectorSubcoreMesh(core_axis_name="c", subcore_axis_name="s")


@functools.partial(
    pl.kernel,
    mesh=_mesh,
    out_type=[jax.ShapeDtypeStruct((N_ROWS, D_MODEL), jnp.float32),
              jax.ShapeDtypeStruct((NW, D_MODEL), jnp.float32)],
    scratch_types=[
        pltpu.VMEM((256,), jnp.float32),
        pltpu.VMEM((RPW,), jnp.int32),
        pltpu.VMEM((RPW,), jnp.int32),
        pltpu.VMEM((RPW,), jnp.int32),
        pltpu.VMEM((RPW,), jnp.int32),
        pltpu.VMEM((RPW + 16,), jnp.float32),
        pltpu.VMEM((RPW + 16,), jnp.float32),
        pltpu.VMEM((CHUNK, 128), jnp.float32),
        pltpu.VMEM((CHUNK, 128), jnp.float32),
        pltpu.VMEM((CHUNK, 128), jnp.float32),
        pltpu.VMEM((CHUNK, 128), jnp.float32),
        pltpu.VMEM((CHUNK, D_MODEL), jnp.float32),
        pltpu.VMEM((1, D_MODEL), jnp.float32),
        pltpu.SemaphoreType.DMA,
    ],
)
def _sc_embed(g_hbm, r_hbm, t_hbm, q_hbm, pv_hbm, hp_hbm,
              gate_hbm, pe_hbm, roleb_hbm, pirow_hbm,
              out_hbm, psum_hbm,
              pirow_v, gb, rb, tb, qb, pvb, hpb,
              grow, trow, qrow, rrow, outbuf, psum_v, sem):
    wid = lax.axis_index("s") * 2 + lax.axis_index("c")
    base = wid * RPW

    pltpu.sync_copy(pirow_hbm, pirow_v)
    pltpu.sync_copy(g_hbm.at[pl.ds(base, RPW)], gb)
    pltpu.sync_copy(r_hbm.at[pl.ds(base, RPW)], rb)
    pltpu.sync_copy(t_hbm.at[pl.ds(base, RPW)], tb)
    pltpu.sync_copy(q_hbm.at[pl.ds(base, RPW)], qb)
    pltpu.sync_copy(pv_hbm.at[pl.ds(base, RPW)], pvb.at[pl.ds(0, RPW)])
    pltpu.sync_copy(hp_hbm.at[pl.ds(base, RPW)], hpb.at[pl.ds(0, RPW)])

    prow_k = [pirow_v[pl.ds(16 * k, 16)] for k in range(8)]
    irow_k = [pirow_v[pl.ds(128 + 16 * k, 16)] for k in range(8)]
    zero16 = jnp.zeros((16,), jnp.float32)

    acc = tuple([zero16] * 16)
    for c in range(RPW // CHUNK):
        o = c * CHUNK
        d1 = pltpu.async_copy(gate_hbm.at[gb.at[pl.ds(o, CHUNK)]], grow, sem)
        d2 = pltpu.async_copy(pe_hbm.at[tb.at[pl.ds(o, CHUNK)]], trow, sem)
        d3 = pltpu.async_copy(pe_hbm.at[qb.at[pl.ds(o, CHUNK)]], qrow, sem)
        d4 = pltpu.async_copy(roleb_hbm.at[rb.at[pl.ds(o, CHUNK)]], rrow, sem)
        d1.wait(); d2.wait(); d3.wait(); d4.wait()

        @plsc.parallel_loop(0, CHUNK, step=1, unroll=4)
        def rowloop(nloc):
            n = o + nloc
            pvf = jnp.full((16,), pvb[pl.ds(n, 16)][0], jnp.float32)
            hpf = jnp.full((16,), hpb[pl.ds(n, 16)][0], jnp.float32)
            for k in range(8):
                v = (grow[nloc, pl.ds(16 * k, 16)]
                     + trow[nloc, pl.ds(16 * k, 16)])
                outbuf[nloc, pl.ds(16 * k, 16)] = v
            for k in range(8):
                v = (rrow[nloc, pl.ds(16 * k, 16)]
                     + qrow[nloc, pl.ds(16 * k, 16)])
                if k >= 4:
                    v = v + pvf * prow_k[k]
                if k == 7:
                    v = v + hpf * irow_k[k]
                outbuf[nloc, pl.ds(128 + 16 * k, 16)] = v

        def sumbody(nloc, acc):
            return tuple(
                acc[k] + outbuf[nloc, pl.ds(16 * k, 16)] for k in range(16))

        acc = lax.fori_loop(0, CHUNK, sumbody, acc)
        pltpu.sync_copy(outbuf, out_hbm.at[pl.ds(base + o, CHUNK)])

    for k in range(16):
        psum_v[0, pl.ds(16 * k, 16)] = acc[k]
    pltpu.sync_copy(psum_v, psum_hbm.at[pl.ds(wid, 1)])


def kernel(gate_idx, role_idx, param_val, has_param, grid_positions,
           gate_table, role_table, W_param, b_param):
    roleb = jnp.zeros((4, 128), jnp.float32)
    roleb = roleb.at[:, 0:64].set(role_table)
    roleb = roleb.at[:, 64:127].set(jnp.broadcast_to(b_param[None, :], (4, 63)))
    pirow = jnp.zeros((256,), jnp.float32)
    pirow = pirow.at[64:127].set(W_param[0])
    pirow = pirow.at[255].set(1.0)

    node_embeddings, psum = _sc_embed(
        gate_idx.astype(jnp.int32),
        role_idx.astype(jnp.int32),
        grid_positions[:, 0].astype(jnp.int32),
        grid_positions[:, 1].astype(jnp.int32),
        param_val, has_param,
        gate_table, jnp.asarray(_PE), roleb, pirow)

    circuit_embedding = psum.sum(axis=0) * (1.0 / N_ROWS)
    return node_embeddings, circuit_embedding
